# baseline (speedup 1.0000x reference)
import functools

import jax
import jax.numpy as jnp
from jax import lax
from jax.experimental import pallas as pl
from jax.experimental.pallas import tpu as pltpu
from jax.experimental.pallas import tpu_sc as plsc

N = 10000
F = 128
E = 320000
NC, NS = 2, 16
NW = NC * NS
CH = 128
NCHUNK = 80
EPW = NCHUNK * CH
EPAD = NW * EPW
ROWS = 10240
RPS = ROWS // NS
DUMP = N
MB = 1000

_mesh = plsc.VectorSubcoreMesh(core_axis_name="c", subcore_axis_name="s")


@functools.partial(
    pl.kernel,
    out_type=jax.ShapeDtypeStruct((NC, ROWS, 16), jnp.float32),
    mesh=_mesh,
    scratch_types=[
        pltpu.VMEM((NCHUNK, CH), jnp.int32),
        pltpu.VMEM((CH, 16), jnp.float32),
        pltpu.VMEM((CH, 16), jnp.float32),
        pltpu.VMEM_SHARED((ROWS, 16), jnp.float32),
    ],
)
def _deg_kernel(dst_hbm, out_hbm, dst_v, zeros_v, ones_v, acc):
    c = lax.axis_index("c")
    s = lax.axis_index("s")
    wid = s * NC + c

    def fill(i, _):
        zeros_v[i, :] = jnp.zeros((16,), jnp.float32)
        ones_v[i, :] = jnp.ones((16,), jnp.float32)
        return 0

    lax.fori_loop(0, CH, fill, 0)
    base = s * RPS
    for k in range(RPS // CH):
        pltpu.sync_copy(zeros_v, acc.at[pl.ds(base + k * CH, CH)])
    plsc.subcore_barrier()

    pltpu.sync_copy(dst_hbm.at[wid], dst_v)

    def body(j, _):
        pltpu.sync_copy(ones_v, acc.at[dst_v.at[j]], add=True)
        return 0

    lax.fori_loop(0, NCHUNK, body, 0)
    plsc.subcore_barrier()
    for k in range(RPS // CH):
        pltpu.sync_copy(acc.at[pl.ds(base + k * CH, CH)],
                        out_hbm.at[c, pl.ds(base + k * CH, CH)])


@functools.partial(
    pl.kernel,
    out_type=jax.ShapeDtypeStruct((NC, ROWS, F), jnp.float32),
    mesh=_mesh,
    scratch_types=[
        pltpu.VMEM((NCHUNK // 2, CH), jnp.int32),
        pltpu.VMEM((NCHUNK // 2, CH), jnp.int32),
        pltpu.VMEM((CH, F), jnp.float32),
        pltpu.VMEM((CH, F), jnp.float32),
        pltpu.VMEM_SHARED((ROWS, F), jnp.float32),
        pltpu.SemaphoreType.DMA,
        pltpu.SemaphoreType.DMA,
    ],
)
def _scatter_kernel(tab_hbm, src_hbm, dst_hbm, out_hbm,
                    src_v, dst_v, rows0_v, rows1_v, acc, sem0, sem1):
    c = lax.axis_index("c")
    s = lax.axis_index("s")
    wid = s * NC + c
    NH = NCHUNK // 2

    def zfill(i, _):
        for l in range(F // 16):
            rows0_v[i, pl.ds(l * 16, 16)] = jnp.zeros((16,), jnp.float32)
        return 0

    lax.fori_loop(0, CH, zfill, 0)
    base = s * RPS
    for k in range(RPS // CH):
        pltpu.sync_copy(rows0_v, acc.at[pl.ds(base + k * CH, CH)])
    plsc.subcore_barrier()

    for h in range(2):
        pltpu.sync_copy(src_hbm.at[wid, h], src_v)
        pltpu.sync_copy(dst_hbm.at[wid, h], dst_v)

        def body(g, _):
            pltpu.async_copy(tab_hbm.at[src_v.at[2 * g]], rows0_v, sem0).wait()
            pltpu.sync_copy(rows0_v, acc.at[dst_v.at[2 * g]], add=True)
            pltpu.async_copy(tab_hbm.at[src_v.at[2 * g + 1]], rows1_v, sem1).wait()
            pltpu.sync_copy(rows1_v, acc.at[dst_v.at[2 * g + 1]], add=True)
            return 0

        lax.fori_loop(0, NH // 2, body, 0)
    plsc.subcore_barrier()
    for k in range(RPS // CH):
        pltpu.sync_copy(acc.at[pl.ds(base + k * CH, CH)],
                        out_hbm.at[c, pl.ds(base + k * CH, CH)])


def _dis(d0_ref, d1_ref):
    deg = d0_ref[:, 0:1] + d1_ref[:, 0:1] + 1.0
    return lax.rsqrt(deg)


def _tc1_body(x_ref, w_ref, d0_ref, d1_ref, o_ref):
    o_ref[...] = jnp.dot(x_ref[...], w_ref[...],
                         preferred_element_type=jnp.float32) * _dis(d0_ref, d1_ref)


def _tc2_body(s0_ref, s1_ref, h1p_ref, d0_ref, d1_ref, w_ref, b_ref, o_ref):
    dis = _dis(d0_ref, d1_ref)
    h = (s0_ref[...] + s1_ref[...] + h1p_ref[...]) * dis + b_ref[...]
    h = jnp.maximum(h, 0.0)
    o_ref[...] = jnp.dot(h, w_ref[...],
                         preferred_element_type=jnp.float32) * dis


def _tc3_body(s0_ref, s1_ref, h2p_ref, d0_ref, d1_ref, b_ref, o_ref):
    dis = _dis(d0_ref, d1_ref)
    o_ref[...] = (s0_ref[...] + s1_ref[...] + h2p_ref[...]) * dis + b_ref[...]


_row_spec = pl.BlockSpec((MB, F), lambda i: (i, 0))
_deg_spec = pl.BlockSpec((MB, 16), lambda i: (i, 0))
_w_spec = pl.BlockSpec((F, F), lambda i: (0, 0))
_b_spec = pl.BlockSpec((1, F), lambda i: (0, 0))
_out_struct = jax.ShapeDtypeStruct((N, F), jnp.float32)

_tc1 = pl.pallas_call(
    _tc1_body, grid=(N // MB,),
    in_specs=[_row_spec, _w_spec, _deg_spec, _deg_spec],
    out_specs=_row_spec, out_shape=_out_struct)

_tc2 = pl.pallas_call(
    _tc2_body, grid=(N // MB,),
    in_specs=[_row_spec, _row_spec, _row_spec, _deg_spec, _deg_spec,
              _w_spec, _b_spec],
    out_specs=_row_spec, out_shape=_out_struct)

_tc3 = pl.pallas_call(
    _tc3_body, grid=(N // MB,),
    in_specs=[_row_spec, _row_spec, _row_spec, _deg_spec, _deg_spec, _b_spec],
    out_specs=_row_spec, out_shape=_out_struct)


def kernel(x, edge_index, W1, b1, W_mu, b_mu, W_ls, b_ls):
    ei = edge_index.astype(jnp.int32)
    pad = EPAD - E
    src_pad = jnp.concatenate([ei[0], jnp.zeros((pad,), jnp.int32)])
    dump = DUMP + jnp.arange(pad, dtype=jnp.int32) % (ROWS - N)
    dst_pad = jnp.concatenate([ei[1], dump])
    dst_p = dst_pad.reshape(NW, NCHUNK, CH)
    src_h = src_pad.reshape(NW, 2, NCHUNK // 2, CH)
    dst_h = dst_pad.reshape(NW, 2, NCHUNK // 2, CH)

    degs = _deg_kernel(dst_p)
    d0 = degs[0, :N]
    d1 = degs[1, :N]

    h1p = _tc1(x, W1, d0, d1)
    s1 = _scatter_kernel(h1p, src_h, dst_h)

    Wcat = jnp.concatenate([W_mu, W_ls], axis=1)
    bcat = jnp.concatenate([b_mu, b_ls]).reshape(1, F)
    h2p = _tc2(s1[0, :N], s1[1, :N], h1p, d0, d1, Wcat, b1.reshape(1, F))
    s2 = _scatter_kernel(h2p, src_h, dst_h)

    outc = _tc3(s2[0, :N], s2[1, :N], h2p, d0, d1, bcat)
    return outc[:, :64], outc[:, 64:]


# speedup vs baseline: 2.3833x; 2.3833x over previous
import functools

import jax
import jax.numpy as jnp
from jax import lax
from jax.experimental import pallas as pl
from jax.experimental.pallas import tpu as pltpu
from jax.experimental.pallas import tpu_sc as plsc

N = 10000
F = 128
E = 320000
NC, NS = 2, 16
NW = NC * NS
CH = 128
NCHT = E // CH
CPW = NCHT // NW
XTRA = NCHT - CPW * NW
NROW = NCHT + 4
ROWS = 10240
RPS = ROWS // NS
MB = 1000

_mesh = plsc.VectorSubcoreMesh(core_axis_name="c", subcore_axis_name="s")


@functools.partial(
    pl.kernel,
    out_type=jax.ShapeDtypeStruct((NC, ROWS, 16), jnp.float32),
    mesh=_mesh,
    scratch_types=[
        pltpu.VMEM((CPW + 10, CH), jnp.int32),
        pltpu.VMEM((CH, 16), jnp.float32),
        pltpu.VMEM((CH, 16), jnp.float32),
        pltpu.VMEM_SHARED((ROWS, 16), jnp.float32),
    ],
)
def _deg_kernel(dst_hbm, out_hbm, dst_v, zeros_v, ones_v, acc):
    c = lax.axis_index("c")
    s = lax.axis_index("s")
    wid = s * NC + c
    n = jnp.where(wid < XTRA, CPW + 1, CPW)
    start = CPW * wid + jnp.minimum(wid, XTRA)
    astart = (start // 8) * 8
    off = start - astart

    def fill(i, _):
        zeros_v[i, :] = jnp.zeros((16,), jnp.float32)
        ones_v[i, :] = jnp.ones((16,), jnp.float32)
        return 0

    lax.fori_loop(0, CH, fill, 0)
    base = s * RPS
    for k in range(RPS // CH):
        pltpu.sync_copy(zeros_v, acc.at[pl.ds(base + k * CH, CH)])
    plsc.subcore_barrier()

    pltpu.sync_copy(dst_hbm.at[pl.ds(astart, CPW + 10)], dst_v)

    def body(j, _):
        pltpu.sync_copy(ones_v, acc.at[dst_v.at[off + j]], add=True)
        return 0

    lax.fori_loop(0, n, body, 0)
    plsc.subcore_barrier()
    for k in range(RPS // CH):
        pltpu.sync_copy(acc.at[pl.ds(base + k * CH, CH)],
                        out_hbm.at[c, pl.ds(base + k * CH, CH)])


@functools.partial(
    pl.kernel,
    out_type=jax.ShapeDtypeStruct((NC, ROWS, F), jnp.float32),
    mesh=_mesh,
    scratch_types=[
        pltpu.VMEM((CPW + 10, CH), jnp.int32),
        pltpu.VMEM((CPW + 10, CH), jnp.int32),
        pltpu.VMEM((CH, F), jnp.float32),
        pltpu.VMEM_SHARED((ROWS, F), jnp.float32),
        pltpu.SemaphoreType.DMA,
    ],
)
def _scatter_kernel(tab_hbm, src_hbm, dst_hbm, out_hbm,
                    src_v, dst_v, rows_v, acc, sem):
    c = lax.axis_index("c")
    s = lax.axis_index("s")
    wid = s * NC + c
    n = jnp.where(wid < XTRA, CPW + 1, CPW)
    start = CPW * wid + jnp.minimum(wid, XTRA)
    astart = (start // 8) * 8
    off = start - astart

    def zfill(i, _):
        for l in range(F // 16):
            rows_v[i, pl.ds(l * 16, 16)] = jnp.zeros((16,), jnp.float32)
        return 0

    lax.fori_loop(0, CH, zfill, 0)
    base = s * RPS
    for k in range(RPS // CH):
        pltpu.sync_copy(rows_v, acc.at[pl.ds(base + k * CH, CH)])
    plsc.subcore_barrier()

    pltpu.sync_copy(src_hbm.at[pl.ds(astart, CPW + 10)], src_v)
    pltpu.sync_copy(dst_hbm.at[pl.ds(astart, CPW + 10)], dst_v)

    def body(j, _):
        pltpu.async_copy(tab_hbm.at[src_v.at[off + j]], rows_v, sem).wait()
        pltpu.sync_copy(rows_v, acc.at[dst_v.at[off + j]], add=True)
        return 0

    lax.fori_loop(0, n, body, 0)
    plsc.subcore_barrier()
    for k in range(RPS // CH):
        pltpu.sync_copy(acc.at[pl.ds(base + k * CH, CH)],
                        out_hbm.at[c, pl.ds(base + k * CH, CH)])


def _dis(d0_ref, d1_ref):
    deg = d0_ref[:, 0:1] + d1_ref[:, 0:1] + 1.0
    return lax.rsqrt(deg)


def _tc1_body(x_ref, w_ref, d0_ref, d1_ref, o_ref):
    o_ref[...] = jnp.dot(x_ref[...], w_ref[...],
                         preferred_element_type=jnp.float32) * _dis(d0_ref, d1_ref)


def _tc2_body(s0_ref, s1_ref, h1p_ref, d0_ref, d1_ref, w_ref, b_ref, o_ref):
    dis = _dis(d0_ref, d1_ref)
    h = (s0_ref[...] + s1_ref[...] + h1p_ref[...]) * dis + b_ref[...]
    h = jnp.maximum(h, 0.0)
    o_ref[...] = jnp.dot(h, w_ref[...],
                         preferred_element_type=jnp.float32) * dis


def _tc3_body(s0_ref, s1_ref, h2p_ref, d0_ref, d1_ref, b_ref, o_ref):
    dis = _dis(d0_ref, d1_ref)
    o_ref[...] = (s0_ref[...] + s1_ref[...] + h2p_ref[...]) * dis + b_ref[...]


_row_spec = pl.BlockSpec((MB, F), lambda i: (i, 0))
_deg_spec = pl.BlockSpec((MB, 16), lambda i: (i, 0))
_w_spec = pl.BlockSpec((F, F), lambda i: (0, 0))
_b_spec = pl.BlockSpec((1, F), lambda i: (0, 0))
_out_struct = jax.ShapeDtypeStruct((N, F), jnp.float32)

_tc1 = pl.pallas_call(
    _tc1_body, grid=(N // MB,),
    in_specs=[_row_spec, _w_spec, _deg_spec, _deg_spec],
    out_specs=_row_spec, out_shape=_out_struct)

_tc2 = pl.pallas_call(
    _tc2_body, grid=(N // MB,),
    in_specs=[_row_spec, _row_spec, _row_spec, _deg_spec, _deg_spec,
              _w_spec, _b_spec],
    out_specs=_row_spec, out_shape=_out_struct)

_tc3 = pl.pallas_call(
    _tc3_body, grid=(N // MB,),
    in_specs=[_row_spec, _row_spec, _row_spec, _deg_spec, _deg_spec, _b_spec],
    out_specs=_row_spec, out_shape=_out_struct)


def kernel(x, edge_index, W1, b1, W_mu, b_mu, W_ls, b_ls):
    ei = edge_index.astype(jnp.int32)
    pad = NROW * CH - E
    src_p = jnp.concatenate(
        [ei[0], jnp.zeros((pad,), jnp.int32)]).reshape(NROW, CH)
    dst_p = jnp.concatenate(
        [ei[1], jnp.zeros((pad,), jnp.int32)]).reshape(NROW, CH)

    degs = _deg_kernel(dst_p)
    d0 = degs[0, :N]
    d1 = degs[1, :N]

    h1p = _tc1(x, W1, d0, d1)
    s1 = _scatter_kernel(h1p, src_p, dst_p)

    Wcat = jnp.concatenate([W_mu, W_ls], axis=1)
    bcat = jnp.concatenate([b_mu, b_ls]).reshape(1, F)
    h2p = _tc2(s1[0, :N], s1[1, :N], h1p, d0, d1, Wcat, b1.reshape(1, F))
    s2 = _scatter_kernel(h2p, src_p, dst_p)

    outc = _tc3(s2[0, :N], s2[1, :N], h2p, d0, d1, bcat)
    return outc[:, :64], outc[:, 64:]


# speedup vs baseline: 2.4908x; 1.0451x over previous
import functools

import jax
import jax.numpy as jnp
from jax import lax
from jax.experimental import pallas as pl
from jax.experimental.pallas import tpu as pltpu
from jax.experimental.pallas import tpu_sc as plsc

N = 10000
F = 128
E = 320000
NC, NS = 2, 16
NW = NC * NS
CH = 128
NCHT = E // CH
CPW = NCHT // NW
XTRA = NCHT - CPW * NW
NROW = NCHT + 4
ROWS = 10240
RPS = ROWS // NS
MB = 1000

_mesh = plsc.VectorSubcoreMesh(core_axis_name="c", subcore_axis_name="s")


@functools.partial(
    pl.kernel,
    out_type=jax.ShapeDtypeStruct((NC, ROWS, 16), jnp.float32),
    mesh=_mesh,
    scratch_types=[
        pltpu.VMEM((CPW + 10, CH), jnp.int32),
        pltpu.VMEM((CH, 16), jnp.float32),
        pltpu.VMEM((CH, 16), jnp.float32),
        pltpu.VMEM_SHARED((ROWS, 16), jnp.float32),
    ],
)
def _deg_kernel(dst_hbm, out_hbm, dst_v, zeros_v, ones_v, acc):
    c = lax.axis_index("c")
    s = lax.axis_index("s")
    wid = s * NC + c
    n = jnp.where(wid < XTRA, CPW + 1, CPW)
    start = CPW * wid + jnp.minimum(wid, XTRA)
    astart = (start // 8) * 8
    off = start - astart

    def fill(i, _):
        zeros_v[i, :] = jnp.zeros((16,), jnp.float32)
        ones_v[i, :] = jnp.ones((16,), jnp.float32)
        return 0

    lax.fori_loop(0, CH, fill, 0)
    base = s * RPS
    for k in range(RPS // CH):
        pltpu.sync_copy(zeros_v, acc.at[pl.ds(base + k * CH, CH)])
    plsc.subcore_barrier()

    pltpu.sync_copy(dst_hbm.at[pl.ds(astart, CPW + 10)], dst_v)

    def body(j, _):
        pltpu.sync_copy(ones_v, acc.at[dst_v.at[off + j]], add=True)
        return 0

    lax.fori_loop(0, n, body, 0)
    plsc.subcore_barrier()
    for k in range(RPS // CH):
        pltpu.sync_copy(acc.at[pl.ds(base + k * CH, CH)],
                        out_hbm.at[c, pl.ds(base + k * CH, CH)])


@functools.partial(
    pl.kernel,
    out_type=jax.ShapeDtypeStruct((NC, ROWS, F), jnp.float32),
    mesh=_mesh,
    scratch_types=[
        pltpu.VMEM((CPW + 10, CH), jnp.int32),
        pltpu.VMEM((CPW + 10, CH), jnp.int32),
        pltpu.VMEM((CH, F), jnp.float32),
        pltpu.VMEM_SHARED((ROWS, F), jnp.float32),
        pltpu.SemaphoreType.DMA,
    ],
)
def _scatter_kernel(tab_hbm, src_hbm, dst_hbm, out_hbm,
                    src_v, dst_v, rows_v, acc, sem):
    c = lax.axis_index("c")
    s = lax.axis_index("s")
    wid = s * NC + c
    n = jnp.where(wid < XTRA, CPW + 1, CPW)
    start = CPW * wid + jnp.minimum(wid, XTRA)
    astart = (start // 8) * 8
    off = start - astart

    def zfill(i, _):
        for l in range(F // 16):
            rows_v[i, pl.ds(l * 16, 16)] = jnp.zeros((16,), jnp.float32)
        return 0

    lax.fori_loop(0, CH, zfill, 0)
    base = s * RPS
    for k in range(RPS // CH):
        pltpu.sync_copy(rows_v, acc.at[pl.ds(base + k * CH, CH)])
    plsc.subcore_barrier()

    pltpu.sync_copy(src_hbm.at[pl.ds(astart, CPW + 10)], src_v)
    pltpu.sync_copy(dst_hbm.at[pl.ds(astart, CPW + 10)], dst_v)

    def body(j, _):
        pltpu.async_copy(tab_hbm.at[src_v.at[off + j]], rows_v, sem).wait()
        pltpu.sync_copy(rows_v, acc.at[dst_v.at[off + j]], add=True)
        return 0

    lax.fori_loop(0, n, body, 0)
    plsc.subcore_barrier()
    for k in range(RPS // CH):
        pltpu.sync_copy(acc.at[pl.ds(base + k * CH, CH)],
                        out_hbm.at[c, pl.ds(base + k * CH, CH)])


def _dis(d_ref):
    deg = d_ref[0, :, 0:1] + d_ref[1, :, 0:1] + 1.0
    return lax.rsqrt(deg)


def _tc1_body(x_ref, w_ref, d_ref, o_ref):
    o_ref[...] = jnp.dot(x_ref[...], w_ref[...],
                         preferred_element_type=jnp.float32) * _dis(d_ref)


def _tc2_body(s_ref, h1p_ref, d_ref, w_ref, b_ref, o_ref):
    dis = _dis(d_ref)
    h = (s_ref[0] + s_ref[1] + h1p_ref[...]) * dis + b_ref[...]
    h = jnp.maximum(h, 0.0)
    o_ref[...] = jnp.dot(h, w_ref[...],
                         preferred_element_type=jnp.float32) * dis


def _tc3_body(s_ref, h2p_ref, d_ref, b_ref, o_ref):
    dis = _dis(d_ref)
    o_ref[...] = (s_ref[0] + s_ref[1] + h2p_ref[...]) * dis + b_ref[...]


_row_spec = pl.BlockSpec((MB, F), lambda i: (i, 0))
_deg_spec = pl.BlockSpec((2, MB, 16), lambda i: (0, i, 0))
_s_spec = pl.BlockSpec((2, MB, F), lambda i: (0, i, 0))
_w_spec = pl.BlockSpec((F, F), lambda i: (0, 0))
_b_spec = pl.BlockSpec((1, F), lambda i: (0, 0))
_out_struct = jax.ShapeDtypeStruct((N, F), jnp.float32)

_tc1 = pl.pallas_call(
    _tc1_body, grid=(N // MB,),
    in_specs=[_row_spec, _w_spec, _deg_spec],
    out_specs=_row_spec, out_shape=_out_struct)

_tc2 = pl.pallas_call(
    _tc2_body, grid=(N // MB,),
    in_specs=[_s_spec, _row_spec, _deg_spec, _w_spec, _b_spec],
    out_specs=_row_spec, out_shape=_out_struct)

_tc3 = pl.pallas_call(
    _tc3_body, grid=(N // MB,),
    in_specs=[_s_spec, _row_spec, _deg_spec, _b_spec],
    out_specs=_row_spec, out_shape=_out_struct)


def kernel(x, edge_index, W1, b1, W_mu, b_mu, W_ls, b_ls):
    ei = edge_index.astype(jnp.int32)
    pad = NROW * CH - E
    src_p = jnp.concatenate(
        [ei[0], jnp.zeros((pad,), jnp.int32)]).reshape(NROW, CH)
    dst_p = jnp.concatenate(
        [ei[1], jnp.zeros((pad,), jnp.int32)]).reshape(NROW, CH)

    degs = _deg_kernel(dst_p)

    h1p = _tc1(x, W1, degs)
    s1 = _scatter_kernel(h1p, src_p, dst_p)

    Wcat = jnp.concatenate([W_mu, W_ls], axis=1)
    bcat = jnp.concatenate([b_mu, b_ls]).reshape(1, F)
    h2p = _tc2(s1, h1p, degs, Wcat, b1.reshape(1, F))
    s2 = _scatter_kernel(h2p, src_p, dst_p)

    outc = _tc3(s2, h2p, degs, bcat)
    return outc[:, :64], outc[:, 64:]


# speedup vs baseline: 2.5792x; 1.0355x over previous
import functools

import jax
import jax.numpy as jnp
from jax import lax
from jax.experimental import pallas as pl
from jax.experimental.pallas import tpu as pltpu
from jax.experimental.pallas import tpu_sc as plsc

N = 10000
F = 128
E = 320000
NC, NS = 2, 16
NW = NC * NS
CH = 128
NCHT = E // CH
CPW = NCHT // NW
XTRA = NCHT - CPW * NW
LD = 88
ROWS = 10240
RPS = ROWS // NS
MB = 1000

_mesh = plsc.VectorSubcoreMesh(core_axis_name="c", subcore_axis_name="s")


@functools.partial(
    pl.kernel,
    out_type=jax.ShapeDtypeStruct((NC, ROWS, 16), jnp.float32),
    mesh=_mesh,
    scratch_types=[
        pltpu.VMEM((LD, CH), jnp.int32),
        pltpu.VMEM((CH, 16), jnp.float32),
        pltpu.VMEM((CH, 16), jnp.float32),
        pltpu.VMEM_SHARED((ROWS, 16), jnp.float32),
    ],
)
def _deg_kernel(ei_hbm, tail_hbm, out_hbm, dst_v, zeros_v, ones_v, acc):
    c = lax.axis_index("c")
    s = lax.axis_index("s")
    wid = s * NC + c
    is_last = wid == NW - 1
    n = jnp.where(wid < XTRA, CPW + 1, CPW)
    start = CPW * wid + jnp.minimum(wid, XTRA)
    astart = (start // 8) * 8
    off = jnp.where(is_last, 0, start - astart)

    def fill(i, _):
        zeros_v[i, :] = jnp.zeros((16,), jnp.float32)
        ones_v[i, :] = jnp.ones((16,), jnp.float32)
        return 0

    lax.fori_loop(0, CH, fill, 0)
    base = s * RPS
    for k in range(RPS // CH):
        pltpu.sync_copy(zeros_v, acc.at[pl.ds(base + k * CH, CH)])
    plsc.subcore_barrier()

    @pl.when(jnp.logical_not(is_last))
    def _():
        pltpu.sync_copy(ei_hbm.at[1, pl.ds(astart, LD)], dst_v)

    @pl.when(is_last)
    def _():
        pltpu.sync_copy(tail_hbm.at[1], dst_v)

    def body(j, _):
        pltpu.sync_copy(ones_v, acc.at[dst_v.at[off + j]], add=True)
        return 0

    lax.fori_loop(0, n, body, 0)
    plsc.subcore_barrier()
    for k in range(RPS // CH):
        pltpu.sync_copy(acc.at[pl.ds(base + k * CH, CH)],
                        out_hbm.at[c, pl.ds(base + k * CH, CH)])


@functools.partial(
    pl.kernel,
    out_type=jax.ShapeDtypeStruct((NC, ROWS, F), jnp.float32),
    mesh=_mesh,
    scratch_types=[
        pltpu.VMEM((LD, CH), jnp.int32),
        pltpu.VMEM((LD, CH), jnp.int32),
        pltpu.VMEM((CH, F), jnp.float32),
        pltpu.VMEM_SHARED((ROWS, F), jnp.float32),
        pltpu.SemaphoreType.DMA,
    ],
)
def _scatter_kernel(tab_hbm, ei_hbm, tail_hbm, out_hbm,
                    src_v, dst_v, rows_v, acc, sem):
    c = lax.axis_index("c")
    s = lax.axis_index("s")
    wid = s * NC + c
    is_last = wid == NW - 1
    n = jnp.where(wid < XTRA, CPW + 1, CPW)
    start = CPW * wid + jnp.minimum(wid, XTRA)
    astart = (start // 8) * 8
    off = jnp.where(is_last, 0, start - astart)

    def zfill(i, _):
        for l in range(F // 16):
            rows_v[i, pl.ds(l * 16, 16)] = jnp.zeros((16,), jnp.float32)
        return 0

    lax.fori_loop(0, CH, zfill, 0)
    base = s * RPS
    for k in range(RPS // CH):
        pltpu.sync_copy(rows_v, acc.at[pl.ds(base + k * CH, CH)])
    plsc.subcore_barrier()

    @pl.when(jnp.logical_not(is_last))
    def _():
        pltpu.sync_copy(ei_hbm.at[0, pl.ds(astart, LD)], src_v)
        pltpu.sync_copy(ei_hbm.at[1, pl.ds(astart, LD)], dst_v)

    @pl.when(is_last)
    def _():
        pltpu.sync_copy(tail_hbm.at[0], src_v)
        pltpu.sync_copy(tail_hbm.at[1], dst_v)

    def body(j, _):
        pltpu.async_copy(tab_hbm.at[src_v.at[off + j]], rows_v, sem).wait()
        pltpu.sync_copy(rows_v, acc.at[dst_v.at[off + j]], add=True)
        return 0

    lax.fori_loop(0, n, body, 0)
    plsc.subcore_barrier()
    for k in range(RPS // CH):
        pltpu.sync_copy(acc.at[pl.ds(base + k * CH, CH)],
                        out_hbm.at[c, pl.ds(base + k * CH, CH)])


def _dis(d_ref):
    deg = d_ref[0, :, 0:1] + d_ref[1, :, 0:1] + 1.0
    return lax.rsqrt(deg)


def _tc1_body(x_ref, w_ref, d_ref, o_ref):
    o_ref[...] = jnp.dot(x_ref[...], w_ref[...],
                         preferred_element_type=jnp.float32) * _dis(d_ref)


def _tc2_body(s_ref, h1p_ref, d_ref, w_ref, b_ref, o_ref):
    dis = _dis(d_ref)
    h = (s_ref[0] + s_ref[1] + h1p_ref[...]) * dis + b_ref[...]
    h = jnp.maximum(h, 0.0)
    o_ref[...] = jnp.dot(h, w_ref[...],
                         preferred_element_type=jnp.float32) * dis


def _tc3_body(s_ref, h2p_ref, d_ref, b_ref, mu_ref, ls_ref):
    dis = _dis(d_ref)
    o = (s_ref[0] + s_ref[1] + h2p_ref[...]) * dis + b_ref[...]
    mu_ref[...] = o[:, :64]
    ls_ref[...] = o[:, 64:]


_row_spec = pl.BlockSpec((MB, F), lambda i: (i, 0))
_deg_spec = pl.BlockSpec((2, MB, 16), lambda i: (0, i, 0))
_s_spec = pl.BlockSpec((2, MB, F), lambda i: (0, i, 0))
_w_spec = pl.BlockSpec((F, F), lambda i: (0, 0))
_b_spec = pl.BlockSpec((1, F), lambda i: (0, 0))
_out_struct = jax.ShapeDtypeStruct((N, F), jnp.float32)

_tc1 = pl.pallas_call(
    _tc1_body, grid=(N // MB,),
    in_specs=[_row_spec, _w_spec, _deg_spec],
    out_specs=_row_spec, out_shape=_out_struct)

_tc2 = pl.pallas_call(
    _tc2_body, grid=(N // MB,),
    in_specs=[_s_spec, _row_spec, _deg_spec, _w_spec, _b_spec],
    out_specs=_row_spec, out_shape=_out_struct)

_half_spec = pl.BlockSpec((MB, 64), lambda i: (i, 0))
_half_struct = jax.ShapeDtypeStruct((N, 64), jnp.float32)
_tc3 = pl.pallas_call(
    _tc3_body, grid=(N // MB,),
    in_specs=[_s_spec, _row_spec, _deg_spec, _b_spec],
    out_specs=[_half_spec, _half_spec], out_shape=[_half_struct, _half_struct])


def kernel(x, edge_index, W1, b1, W_mu, b_mu, W_ls, b_ls):
    ei = edge_index.astype(jnp.int32).reshape(2, NCHT, CH)
    last_start = CPW * (NW - 1) + XTRA
    tail = jnp.pad(ei[:, last_start:, :],
                   ((0, 0), (0, LD - (NCHT - last_start)), (0, 0)))

    degs = _deg_kernel(ei, tail)

    h1p = _tc1(x, W1, degs)
    s1 = _scatter_kernel(h1p, ei, tail)

    Wcat = jnp.concatenate([W_mu, W_ls], axis=1)
    bcat = jnp.concatenate([b_mu, b_ls]).reshape(1, F)
    h2p = _tc2(s1, h1p, degs, Wcat, b1.reshape(1, F))
    s2 = _scatter_kernel(h2p, ei, tail)

    mu, ls = _tc3(s2, h2p, degs, bcat)
    return mu, ls
